# baseline (speedup 1.0000x reference)
import functools

import jax
import jax.numpy as jnp
from jax import lax
from jax.experimental import pallas as pl
from jax.experimental.pallas import tpu as pltpu
from jax.experimental.pallas import tpu_sc as plsc

_LANES = 16
_NSUB = 16
_CROWS = 40


def _sc_gather_body(rows_per_sub, beta_hbm, bias_hbm, y_hbm, dep_hbm, out_hbm,
                    table_v, idx0, idx1, val0, val1,
                    si0, si1, so0, so1):
    del dep_hbm
    cid = lax.axis_index("c")
    sid = lax.axis_index("s")

    @pl.when(cid == 0)
    def _():
        pltpu.sync_copy(beta_hbm, table_v)

    @pl.when(cid != 0)
    def _():
        pltpu.sync_copy(bias_hbm, table_v)

    base = sid * rows_per_sub
    nch = rows_per_sub // _CROWS
    idx_b = (idx0, idx1)
    val_b = (val0, val1)
    si_b = (si0, si1)
    so_b = (so0, so1)

    def start_in(ch, b):
        pltpu.async_copy(y_hbm.at[pl.ds(base + ch * _CROWS, _CROWS), :],
                         idx_b[b], si_b[b])

    def wait_in(ch, b):
        pltpu.make_async_copy(y_hbm.at[pl.ds(base + ch * _CROWS, _CROWS), :],
                              idx_b[b], si_b[b]).wait()

    def start_out(ch, b):
        pltpu.async_copy(val_b[b],
                         out_hbm.at[cid, pl.ds(base + ch * _CROWS, _CROWS), :],
                         so_b[b])

    def wait_out(ch, b):
        pltpu.make_async_copy(
            val_b[b],
            out_hbm.at[cid, pl.ds(base + ch * _CROWS, _CROWS), :],
            so_b[b]).wait()

    def gather(b):
        iv_ref = idx_b[b]
        ov_ref = val_b[b]

        @plsc.parallel_loop(0, _CROWS, step=1, unroll=2)
        def _(r):
            ivs = [iv_ref[r, pl.ds(c * _LANES, _LANES)]
                   for c in range(128 // _LANES)]
            gs = [plsc.load_gather(table_v, [iv]) for iv in ivs]
            for c in range(128 // _LANES):
                ov_ref[r, pl.ds(c * _LANES, _LANES)] = gs[c]

    start_in(0, 0)

    def body(h, _):
        c0 = 2 * h
        c1 = c0 + 1
        wait_in(c0, 0)
        start_in(c1, 1)

        @pl.when(h > 0)
        def _():
            wait_out(c0 - 2, 0)

        gather(0)
        start_out(c0, 0)
        wait_in(c1, 1)

        @pl.when(c1 + 1 < nch)
        def _():
            start_in(c1 + 1, 0)

        @pl.when(h > 0)
        def _():
            wait_out(c1 - 2, 1)

        gather(1)
        start_out(c1, 1)
        return 0

    lax.fori_loop(0, nch // 2, body, 0)
    wait_out(nch - 2, 0)
    wait_out(nch - 1, 1)


@functools.partial(jax.jit, static_argnames=("n_tab",))
def _sc_gather(beta_f, bias_f, y2, dep, *, n_tab):
    rows = y2.shape[0]
    rows_per_sub = rows // _NSUB
    body = functools.partial(_sc_gather_body, rows_per_sub)
    return pl.kernel(
        body,
        out_type=jax.ShapeDtypeStruct((2, rows, 128), jnp.float32),
        mesh=plsc.VectorSubcoreMesh(core_axis_name="c", subcore_axis_name="s"),
        scratch_types=[
            pltpu.VMEM((n_tab,), jnp.float32),
            pltpu.VMEM((_CROWS, 128), jnp.int32),
            pltpu.VMEM((_CROWS, 128), jnp.int32),
            pltpu.VMEM((_CROWS, 128), jnp.float32),
            pltpu.VMEM((_CROWS, 128), jnp.float32),
            pltpu.SemaphoreType.DMA,
            pltpu.SemaphoreType.DMA,
            pltpu.SemaphoreType.DMA,
            pltpu.SemaphoreType.DMA,
        ],
        compiler_params=pltpu.CompilerParams(needs_layout_passes=False),
        name="sc_pair_gather",
    )(beta_f, bias_f, y2, dep)


def _combine_body(x_ref, g_ref, o_ref):
    xv = x_ref[...]
    bg = g_ref[0]
    bb = g_ref[1]
    t = jnp.log1p(xv) * bg + bb
    o_ref[...] = jax.nn.sigmoid(t) - jax.nn.sigmoid(bb)


@functools.partial(jax.jit, static_argnames=("bm",))
def _tc_combine(x2, g3, *, bm):
    m = x2.shape[0]
    return pl.pallas_call(
        _combine_body,
        grid=(m // bm,),
        in_specs=[
            pl.BlockSpec((bm, 128), lambda i: (i, 0)),
            pl.BlockSpec((2, bm, 128), lambda i: (0, i, 0)),
        ],
        out_specs=pl.BlockSpec((bm, 128), lambda i: (i, 0)),
        out_shape=jax.ShapeDtypeStruct((m, 128), jnp.float32),
        name="tc_logsigm_combine",
    )(x2, g3)


def kernel(x, y, beta, bias):
    b, c = x.shape
    n = b * c
    n_tab = beta.shape[1]
    m = n // 128
    h = m // 2
    y2 = y.astype(jnp.int32).reshape(m, 128)
    x2 = x.reshape(m, 128)
    beta_f = beta.reshape(-1)
    bias_f = bias.reshape(-1)
    g_a = _sc_gather(beta_f, bias_f, y2[:h], jnp.zeros((8,), jnp.float32),
                     n_tab=n_tab)
    g_b = _sc_gather(beta_f, bias_f, y2[h:], g_a[0, 0, 0:8], n_tab=n_tab)
    out_a = _tc_combine(x2[:h], g_a, bm=1024)
    out_b = _tc_combine(x2[h:], g_b, bm=1024)
    return jnp.concatenate([out_a, out_b], axis=0).reshape(b, c)


# speedup vs baseline: 1.2768x; 1.2768x over previous
import functools

import jax
import jax.numpy as jnp
from jax import lax
from jax.experimental import pallas as pl
from jax.experimental.pallas import tpu as pltpu
from jax.experimental.pallas import tpu_sc as plsc

_LANES = 16
_NSUB = 16
_CROWS = 40


def _sc_gather_body(rows_per_sub, beta_hbm, bias_hbm, y_hbm, out_hbm,
                    table_v, idx0, idx1, val0, val1,
                    si0, si1, so0, so1):
    cid = lax.axis_index("c")
    sid = lax.axis_index("s")

    @pl.when(cid == 0)
    def _():
        pltpu.sync_copy(beta_hbm, table_v)

    @pl.when(cid != 0)
    def _():
        pltpu.sync_copy(bias_hbm, table_v)

    base = sid * rows_per_sub
    nch = rows_per_sub // _CROWS
    idx_b = (idx0, idx1)
    val_b = (val0, val1)
    si_b = (si0, si1)
    so_b = (so0, so1)

    def start_in(ch, b):
        pltpu.async_copy(
            y_hbm.at[pl.ds((base + ch * _CROWS) * 128, _CROWS * 128)],
            idx_b[b], si_b[b])

    def wait_in(ch, b):
        pltpu.make_async_copy(
            y_hbm.at[pl.ds((base + ch * _CROWS) * 128, _CROWS * 128)],
            idx_b[b], si_b[b]).wait()

    def start_out(ch, b):
        pltpu.async_copy(val_b[b],
                         out_hbm.at[cid, pl.ds(base + ch * _CROWS, _CROWS), :],
                         so_b[b])

    def wait_out(ch, b):
        pltpu.make_async_copy(
            val_b[b],
            out_hbm.at[cid, pl.ds(base + ch * _CROWS, _CROWS), :],
            so_b[b]).wait()

    def gather(b):
        iv_ref = idx_b[b]
        ov_ref = val_b[b]

        @plsc.parallel_loop(0, _CROWS, step=1, unroll=2)
        def _(r):
            ivs = [iv_ref[pl.ds(r * 128 + c * _LANES, _LANES)]
                   for c in range(128 // _LANES)]
            gs = [plsc.load_gather(table_v, [iv]) for iv in ivs]
            for c in range(128 // _LANES):
                ov_ref[r, pl.ds(c * _LANES, _LANES)] = gs[c]

    start_in(0, 0)

    def body(h, _):
        c0 = 2 * h
        c1 = c0 + 1
        wait_in(c0, 0)
        start_in(c1, 1)

        @pl.when(h > 0)
        def _():
            wait_out(c0 - 2, 0)

        gather(0)
        start_out(c0, 0)
        wait_in(c1, 1)

        @pl.when(c1 + 1 < nch)
        def _():
            start_in(c1 + 1, 0)

        @pl.when(h > 0)
        def _():
            wait_out(c1 - 2, 1)

        gather(1)
        start_out(c1, 1)
        return 0

    lax.fori_loop(0, nch // 2, body, 0)
    wait_out(nch - 2, 0)
    wait_out(nch - 1, 1)


@functools.partial(jax.jit, static_argnames=("n_tab",))
def _sc_gather(beta_f, bias_f, y_f, *, n_tab):
    rows = y_f.shape[0] // 128
    rows_per_sub = rows // _NSUB
    body = functools.partial(_sc_gather_body, rows_per_sub)
    return pl.kernel(
        body,
        out_type=jax.ShapeDtypeStruct((2, rows, 128), jnp.float32),
        mesh=plsc.VectorSubcoreMesh(core_axis_name="c", subcore_axis_name="s"),
        scratch_types=[
            pltpu.VMEM((n_tab,), jnp.float32),
            pltpu.VMEM((_CROWS * 128,), jnp.int32),
            pltpu.VMEM((_CROWS * 128,), jnp.int32),
            pltpu.VMEM((_CROWS, 128), jnp.float32),
            pltpu.VMEM((_CROWS, 128), jnp.float32),
            pltpu.SemaphoreType.DMA,
            pltpu.SemaphoreType.DMA,
            pltpu.SemaphoreType.DMA,
            pltpu.SemaphoreType.DMA,
        ],
        compiler_params=pltpu.CompilerParams(needs_layout_passes=False),
        name="sc_pair_gather",
    )(beta_f, bias_f, y_f)


def _combine_body(x_ref, g_ref, o_ref):
    xv = x_ref[...]
    bg = g_ref[0]
    bb = g_ref[1]
    t = jnp.log1p(xv) * bg + bb
    o_ref[...] = jax.nn.sigmoid(t) - jax.nn.sigmoid(bb)


@functools.partial(jax.jit, static_argnames=("bm",))
def _tc_combine(x2, g3, *, bm):
    m = x2.shape[0]
    return pl.pallas_call(
        _combine_body,
        grid=(m // bm,),
        in_specs=[
            pl.BlockSpec((bm, 128), lambda i: (i, 0)),
            pl.BlockSpec((2, bm, 128), lambda i: (0, i, 0)),
        ],
        out_specs=pl.BlockSpec((bm, 128), lambda i: (i, 0)),
        out_shape=jax.ShapeDtypeStruct((m, 128), jnp.float32),
        name="tc_logsigm_combine",
    )(x2, g3)


def kernel(x, y, beta, bias):
    b, c = x.shape
    n = b * c
    n_tab = beta.shape[1]
    m = n // 128
    h = m // 2
    yf = y.astype(jnp.int32).reshape(n)
    x2 = x.reshape(m, 128)
    beta_f = beta.reshape(-1)
    bias_f = bias.reshape(-1)
    del h
    g3 = _sc_gather(beta_f, bias_f, yf, n_tab=n_tab)
    out = _tc_combine(x2, g3, bm=1024)
    return out.reshape(b, c)


# speedup vs baseline: 1.5407x; 1.2067x over previous
import functools

import jax
import jax.numpy as jnp
from jax import lax
from jax.experimental import pallas as pl
from jax.experimental.pallas import tpu as pltpu
from jax.experimental.pallas import tpu_sc as plsc

_LANES = 16
_NSUB = 16
_CROWS = 16
_GOFFS = tuple(range(0, 192, _LANES)) + (200 - _LANES,)


def _sc_gather_body(rows_per_sub, ncols, beta_hbm, bias_hbm, y_hbm, out_hbm,
                    table_v, idx0, idx1, val0, val1,
                    si0, si1, so0, so1):
    cid = lax.axis_index("c")
    sid = lax.axis_index("s")

    @pl.when(cid == 0)
    def _():
        pltpu.sync_copy(beta_hbm, table_v)

    @pl.when(cid != 0)
    def _():
        pltpu.sync_copy(bias_hbm, table_v)

    base = sid * rows_per_sub
    nch = rows_per_sub // _CROWS
    idx_b = (idx0, idx1)
    val_b = (val0, val1)
    si_b = (si0, si1)
    so_b = (so0, so1)

    def start_in(ch, b):
        pltpu.async_copy(y_hbm.at[pl.ds(base + ch * _CROWS, _CROWS), :],
                         idx_b[b], si_b[b])

    def wait_in(ch, b):
        pltpu.make_async_copy(y_hbm.at[pl.ds(base + ch * _CROWS, _CROWS), :],
                              idx_b[b], si_b[b]).wait()

    def start_out(ch, b):
        pltpu.async_copy(val_b[b],
                         out_hbm.at[cid, pl.ds(base + ch * _CROWS, _CROWS), :],
                         so_b[b])

    def wait_out(ch, b):
        pltpu.make_async_copy(
            val_b[b],
            out_hbm.at[cid, pl.ds(base + ch * _CROWS, _CROWS), :],
            so_b[b]).wait()

    def gather(b):
        iv_ref = idx_b[b]
        ov_ref = val_b[b]

        @plsc.parallel_loop(0, _CROWS, step=1, unroll=2)
        def _(r):
            ivs = [iv_ref[r, pl.ds(c, _LANES)] for c in _GOFFS]
            gs = [plsc.load_gather(table_v, [iv]) for iv in ivs]
            for k, c in enumerate(_GOFFS):
                ov_ref[r, pl.ds(c, _LANES)] = gs[k]

    start_in(0, 0)

    def body(h, _):
        c0 = 2 * h
        c1 = c0 + 1
        wait_in(c0, 0)
        start_in(c1, 1)

        @pl.when(h > 0)
        def _():
            wait_out(c0 - 2, 0)

        gather(0)
        start_out(c0, 0)
        wait_in(c1, 1)

        @pl.when(c1 + 1 < nch)
        def _():
            start_in(c1 + 1, 0)

        @pl.when(h > 0)
        def _():
            wait_out(c1 - 2, 1)

        gather(1)
        start_out(c1, 1)
        return 0

    lax.fori_loop(0, nch // 2, body, 0)
    wait_out(nch - 2, 0)
    wait_out(nch - 1, 1)


@functools.partial(jax.jit, static_argnames=("n_tab",))
def _sc_gather(beta_f, bias_f, y2, *, n_tab):
    rows, ncols = y2.shape
    rows_per_sub = rows // _NSUB
    body = functools.partial(_sc_gather_body, rows_per_sub, ncols)
    return pl.kernel(
        body,
        out_type=jax.ShapeDtypeStruct((2, rows, ncols), jnp.float32),
        mesh=plsc.VectorSubcoreMesh(core_axis_name="c", subcore_axis_name="s"),
        scratch_types=[
            pltpu.VMEM((n_tab,), jnp.float32),
            pltpu.VMEM((_CROWS, ncols), jnp.int32),
            pltpu.VMEM((_CROWS, ncols), jnp.int32),
            pltpu.VMEM((_CROWS, ncols), jnp.float32),
            pltpu.VMEM((_CROWS, ncols), jnp.float32),
            pltpu.SemaphoreType.DMA,
            pltpu.SemaphoreType.DMA,
            pltpu.SemaphoreType.DMA,
            pltpu.SemaphoreType.DMA,
        ],
        compiler_params=pltpu.CompilerParams(needs_layout_passes=False),
        name="sc_pair_gather",
    )(beta_f, bias_f, y2)


def _combine_body(x_ref, g_ref, o_ref):
    xv = x_ref[...]
    bg = g_ref[0]
    bb = g_ref[1]
    t = jnp.log1p(xv) * bg + bb
    o_ref[...] = jax.nn.sigmoid(t) - jax.nn.sigmoid(bb)


@functools.partial(jax.jit, static_argnames=("bm",))
def _tc_combine(x, g, *, bm):
    rows, ncols = x.shape
    return pl.pallas_call(
        _combine_body,
        grid=(rows // bm,),
        in_specs=[
            pl.BlockSpec((bm, ncols), lambda i: (i, 0)),
            pl.BlockSpec((2, bm, ncols), lambda i: (0, i, 0)),
        ],
        out_specs=pl.BlockSpec((bm, ncols), lambda i: (i, 0)),
        out_shape=jax.ShapeDtypeStruct((rows, ncols), jnp.float32),
        name="tc_logsigm_combine",
    )(x, g)


def kernel(x, y, beta, bias):
    n_tab = beta.shape[1]
    y2 = y.astype(jnp.int32)
    g = _sc_gather(beta.reshape(-1), bias.reshape(-1), y2, n_tab=n_tab)
    return _tc_combine(x, g, bm=1024)


# speedup vs baseline: 1.5428x; 1.0014x over previous
import functools

import jax
import jax.numpy as jnp
from jax import lax
from jax.experimental import pallas as pl
from jax.experimental.pallas import tpu as pltpu
from jax.experimental.pallas import tpu_sc as plsc

_LANES = 16
_NSUB = 16
_CROWS = 16
_GOFFS = tuple(range(0, 192, _LANES)) + (200 - _LANES,)


def _sc_gather_body(rows_per_sub, ncols, beta_hbm, bias_hbm, y_hbm, out_hbm,
                    table_v, idx0, idx1, val0, val1,
                    si0, si1, so0, so1):
    cid = lax.axis_index("c")
    sid = lax.axis_index("s")

    @pl.when(cid == 0)
    def _():
        pltpu.sync_copy(beta_hbm, table_v)

    @pl.when(cid != 0)
    def _():
        pltpu.sync_copy(bias_hbm, table_v)

    base = sid * rows_per_sub
    nch = rows_per_sub // _CROWS
    idx_b = (idx0, idx1)
    val_b = (val0, val1)
    si_b = (si0, si1)
    so_b = (so0, so1)

    def start_in(ch, b):
        pltpu.async_copy(y_hbm.at[pl.ds(base + ch * _CROWS, _CROWS), :],
                         idx_b[b], si_b[b])

    def wait_in(ch, b):
        pltpu.make_async_copy(y_hbm.at[pl.ds(base + ch * _CROWS, _CROWS), :],
                              idx_b[b], si_b[b]).wait()

    def start_out(ch, b):
        pltpu.async_copy(val_b[b],
                         out_hbm.at[cid, pl.ds(base + ch * _CROWS, _CROWS), :],
                         so_b[b])

    def wait_out(ch, b):
        pltpu.make_async_copy(
            val_b[b],
            out_hbm.at[cid, pl.ds(base + ch * _CROWS, _CROWS), :],
            so_b[b]).wait()

    def gather(b):
        iv_ref = idx_b[b]
        ov_ref = val_b[b]

        @plsc.parallel_loop(0, _CROWS, step=1, unroll=2)
        def _(r):
            ivs = [iv_ref[r, pl.ds(c, _LANES)] for c in _GOFFS]
            gs = [plsc.load_gather(table_v, [iv]) for iv in ivs]
            for k, c in enumerate(_GOFFS):
                ov_ref[r, pl.ds(c, _LANES)] = gs[k]

    start_in(0, 0)

    def body(h, _):
        c0 = 2 * h
        c1 = c0 + 1
        wait_in(c0, 0)
        start_in(c1, 1)

        @pl.when(h > 0)
        def _():
            wait_out(c0 - 2, 0)

        gather(0)
        start_out(c0, 0)
        wait_in(c1, 1)

        @pl.when(c1 + 1 < nch)
        def _():
            start_in(c1 + 1, 0)

        @pl.when(h > 0)
        def _():
            wait_out(c1 - 2, 1)

        gather(1)
        start_out(c1, 1)
        return 0

    lax.fori_loop(0, nch // 2, body, 0)
    wait_out(nch - 2, 0)
    wait_out(nch - 1, 1)


@functools.partial(jax.jit, static_argnames=("n_tab",))
def _sc_gather(beta_f, bias_f, y2, *, n_tab):
    rows, ncols = y2.shape
    rows_per_sub = rows // _NSUB
    body = functools.partial(_sc_gather_body, rows_per_sub, ncols)
    return pl.kernel(
        body,
        out_type=jax.ShapeDtypeStruct((2, rows, ncols), jnp.float32),
        mesh=plsc.VectorSubcoreMesh(core_axis_name="c", subcore_axis_name="s"),
        scratch_types=[
            pltpu.VMEM((n_tab,), jnp.float32),
            pltpu.VMEM((_CROWS, ncols), jnp.int32),
            pltpu.VMEM((_CROWS, ncols), jnp.int32),
            pltpu.VMEM((_CROWS, ncols), jnp.float32),
            pltpu.VMEM((_CROWS, ncols), jnp.float32),
            pltpu.SemaphoreType.DMA,
            pltpu.SemaphoreType.DMA,
            pltpu.SemaphoreType.DMA,
            pltpu.SemaphoreType.DMA,
        ],
        compiler_params=pltpu.CompilerParams(needs_layout_passes=False,
                                             use_tc_tiling_on_sc=True),
        name="sc_pair_gather",
    )(beta_f, bias_f, y2)


def _combine_body(x_ref, g_ref, o_ref):
    xv = x_ref[...]
    bg = g_ref[0]
    bb = g_ref[1]
    t = jnp.log1p(xv) * bg + bb
    o_ref[...] = jax.nn.sigmoid(t) - jax.nn.sigmoid(bb)


@functools.partial(jax.jit, static_argnames=("bm",))
def _tc_combine(x, g, *, bm):
    rows, ncols = x.shape
    return pl.pallas_call(
        _combine_body,
        grid=(rows // bm,),
        in_specs=[
            pl.BlockSpec((bm, ncols), lambda i: (i, 0)),
            pl.BlockSpec((2, bm, ncols), lambda i: (0, i, 0)),
        ],
        out_specs=pl.BlockSpec((bm, ncols), lambda i: (i, 0)),
        out_shape=jax.ShapeDtypeStruct((rows, ncols), jnp.float32),
        name="tc_logsigm_combine",
    )(x, g)


def kernel(x, y, beta, bias):
    n_tab = beta.shape[1]
    y2 = y.astype(jnp.int32)
    g = _sc_gather(beta.reshape(-1), bias.reshape(-1), y2, n_tab=n_tab)
    return _tc_combine(x, g, bm=1024)


# speedup vs baseline: 1.5565x; 1.0088x over previous
import functools

import jax
import jax.numpy as jnp
from jax import lax
from jax.experimental import pallas as pl
from jax.experimental.pallas import tpu as pltpu
from jax.experimental.pallas import tpu_sc as plsc

_LANES = 16
_NSUB = 16
_CROWS = 32
_GOFFS = tuple(range(0, 192, _LANES)) + (200 - _LANES,)


def _sc_gather_body(rows_per_sub, ncols, beta_hbm, bias_hbm, y_hbm, out_hbm,
                    table_v, idx0, idx1, val0, val1,
                    si0, si1, so0, so1):
    cid = lax.axis_index("c")
    sid = lax.axis_index("s")

    @pl.when(cid == 0)
    def _():
        pltpu.sync_copy(beta_hbm, table_v)

    @pl.when(cid != 0)
    def _():
        pltpu.sync_copy(bias_hbm, table_v)

    base = sid * rows_per_sub
    nch = rows_per_sub // _CROWS
    idx_b = (idx0, idx1)
    val_b = (val0, val1)
    si_b = (si0, si1)
    so_b = (so0, so1)

    def start_in(ch, b):
        pltpu.async_copy(
            y_hbm.at[pl.ds((base + ch * _CROWS) * ncols, _CROWS * ncols)],
            idx_b[b], si_b[b])

    def wait_in(ch, b):
        pltpu.make_async_copy(
            y_hbm.at[pl.ds((base + ch * _CROWS) * ncols, _CROWS * ncols)],
            idx_b[b], si_b[b]).wait()

    def start_out(ch, b):
        pltpu.async_copy(val_b[b],
                         out_hbm.at[cid, pl.ds(base + ch * _CROWS, _CROWS), :],
                         so_b[b])

    def wait_out(ch, b):
        pltpu.make_async_copy(
            val_b[b],
            out_hbm.at[cid, pl.ds(base + ch * _CROWS, _CROWS), :],
            so_b[b]).wait()

    def gather(b):
        iv_ref = idx_b[b]
        ov_ref = val_b[b]

        @plsc.parallel_loop(0, _CROWS, step=1, unroll=2)
        def _(r):
            ivs = [iv_ref[pl.ds(r * ncols + c, _LANES)] for c in _GOFFS]
            gs = [plsc.load_gather(table_v, [iv]) for iv in ivs]
            for k, c in enumerate(_GOFFS):
                ov_ref[r, pl.ds(c, _LANES)] = gs[k]

    start_in(0, 0)

    def body(h, _):
        c0 = 2 * h
        c1 = c0 + 1
        wait_in(c0, 0)
        start_in(c1, 1)

        @pl.when(h > 0)
        def _():
            wait_out(c0 - 2, 0)

        gather(0)
        start_out(c0, 0)
        wait_in(c1, 1)

        @pl.when(c1 + 1 < nch)
        def _():
            start_in(c1 + 1, 0)

        @pl.when(h > 0)
        def _():
            wait_out(c1 - 2, 1)

        gather(1)
        start_out(c1, 1)
        return 0

    lax.fori_loop(0, nch // 2, body, 0)
    wait_out(nch - 2, 0)
    wait_out(nch - 1, 1)


@functools.partial(jax.jit, static_argnames=("n_tab", "ncols"))
def _sc_gather(beta_f, bias_f, y_f, *, n_tab, ncols):
    rows = y_f.shape[0] // ncols
    rows_per_sub = rows // _NSUB
    body = functools.partial(_sc_gather_body, rows_per_sub, ncols)
    return pl.kernel(
        body,
        out_type=jax.ShapeDtypeStruct((2, rows, ncols), jnp.float32),
        mesh=plsc.VectorSubcoreMesh(core_axis_name="c", subcore_axis_name="s"),
        scratch_types=[
            pltpu.VMEM((n_tab,), jnp.float32),
            pltpu.VMEM((_CROWS * ncols,), jnp.int32),
            pltpu.VMEM((_CROWS * ncols,), jnp.int32),
            pltpu.VMEM((_CROWS, ncols), jnp.float32),
            pltpu.VMEM((_CROWS, ncols), jnp.float32),
            pltpu.SemaphoreType.DMA,
            pltpu.SemaphoreType.DMA,
            pltpu.SemaphoreType.DMA,
            pltpu.SemaphoreType.DMA,
        ],
        compiler_params=pltpu.CompilerParams(needs_layout_passes=False),
        name="sc_pair_gather",
    )(beta_f, bias_f, y_f)


def _combine_body(x_ref, g_ref, o_ref):
    xv = x_ref[...]
    bg = g_ref[0]
    bb = g_ref[1]
    t = jnp.log1p(xv) * bg + bb
    o_ref[...] = jax.nn.sigmoid(t) - jax.nn.sigmoid(bb)


@functools.partial(jax.jit, static_argnames=("bm",))
def _tc_combine(x, g, *, bm):
    rows, ncols = x.shape
    return pl.pallas_call(
        _combine_body,
        grid=(rows // bm,),
        in_specs=[
            pl.BlockSpec((bm, ncols), lambda i: (i, 0)),
            pl.BlockSpec((2, bm, ncols), lambda i: (0, i, 0)),
        ],
        out_specs=pl.BlockSpec((bm, ncols), lambda i: (i, 0)),
        out_shape=jax.ShapeDtypeStruct((rows, ncols), jnp.float32),
        name="tc_logsigm_combine",
    )(x, g)


def kernel(x, y, beta, bias):
    b, c = x.shape
    n_tab = beta.shape[1]
    yf = y.astype(jnp.int32).reshape(b * c)
    g = _sc_gather(beta.reshape(-1), bias.reshape(-1), yf,
                   n_tab=n_tab, ncols=c)
    return _tc_combine(x, g, bm=1024)


# speedup vs baseline: 1.7191x; 1.1045x over previous
import functools

import jax
import jax.numpy as jnp
from jax import lax
from jax.experimental import pallas as pl
from jax.experimental.pallas import tpu as pltpu
from jax.experimental.pallas import tpu_sc as plsc

_LANES = 16
_NSUB = 16
_CROWS = 32
_GOFFS = tuple(range(0, 192, _LANES)) + (200 - _LANES,)


def _sc_gather_body(rows_per_sub, ncols, beta_hbm, bias_hbm, y_hbm, out_hbm,
                    table_v, idx0, idx1, val0, val1,
                    si0, si1, so0, so1):
    cid = lax.axis_index("c")
    sid = lax.axis_index("s")

    @pl.when(cid == 0)
    def _():
        pltpu.sync_copy(beta_hbm, table_v)

    @pl.when(cid != 0)
    def _():
        pltpu.sync_copy(bias_hbm, table_v)

    base = sid * rows_per_sub
    nch = rows_per_sub // _CROWS
    idx_b = (idx0, idx1)
    val_b = (val0, val1)
    si_b = (si0, si1)
    so_b = (so0, so1)

    def start_in(ch, b):
        pltpu.async_copy(
            y_hbm.at[pl.ds((base + ch * _CROWS) * ncols, _CROWS * ncols)],
            idx_b[b], si_b[b])

    def wait_in(ch, b):
        pltpu.make_async_copy(
            y_hbm.at[pl.ds((base + ch * _CROWS) * ncols, _CROWS * ncols)],
            idx_b[b], si_b[b]).wait()

    def start_out(ch, b):
        pltpu.async_copy(val_b[b],
                         out_hbm.at[cid, pl.ds(base + ch * _CROWS, _CROWS), :],
                         so_b[b])

    def wait_out(ch, b):
        pltpu.make_async_copy(
            val_b[b],
            out_hbm.at[cid, pl.ds(base + ch * _CROWS, _CROWS), :],
            so_b[b]).wait()

    def gather(b):
        iv_ref = idx_b[b]
        ov_ref = val_b[b]

        @plsc.parallel_loop(0, _CROWS, step=1, unroll=2)
        def _(r):
            ivs = [iv_ref[pl.ds(r * ncols + c, _LANES)] for c in _GOFFS]
            gs = [plsc.load_gather(table_v, [iv]) for iv in ivs]
            for k, c in enumerate(_GOFFS):
                ov_ref[r, pl.ds(c, _LANES)] = gs[k]

    start_in(0, 0)

    def body(h, _):
        c0 = 2 * h
        c1 = c0 + 1
        wait_in(c0, 0)
        start_in(c1, 1)

        @pl.when(h > 0)
        def _():
            wait_out(c0 - 2, 0)

        gather(0)
        start_out(c0, 0)
        wait_in(c1, 1)

        @pl.when(c1 + 1 < nch)
        def _():
            start_in(c1 + 1, 0)

        @pl.when(h > 0)
        def _():
            wait_out(c1 - 2, 1)

        gather(1)
        start_out(c1, 1)
        return 0

    lax.fori_loop(0, nch // 2, body, 0)
    wait_out(nch - 2, 0)
    wait_out(nch - 1, 1)


@functools.partial(jax.jit, static_argnames=("n_tab", "ncols"))
def _sc_gather(beta_f, bias_f, y_f, *, n_tab, ncols):
    rows = y_f.shape[0] // ncols
    rows_per_sub = rows // _NSUB
    body = functools.partial(_sc_gather_body, rows_per_sub, ncols)
    return pl.kernel(
        body,
        out_type=jax.ShapeDtypeStruct((2, rows, ncols), jnp.float32),
        mesh=plsc.VectorSubcoreMesh(core_axis_name="c", subcore_axis_name="s"),
        scratch_types=[
            pltpu.VMEM((n_tab,), jnp.float32),
            pltpu.VMEM((_CROWS * ncols,), jnp.int32),
            pltpu.VMEM((_CROWS * ncols,), jnp.int32),
            pltpu.VMEM((_CROWS, ncols), jnp.float32),
            pltpu.VMEM((_CROWS, ncols), jnp.float32),
            pltpu.SemaphoreType.DMA,
            pltpu.SemaphoreType.DMA,
            pltpu.SemaphoreType.DMA,
            pltpu.SemaphoreType.DMA,
        ],
        compiler_params=pltpu.CompilerParams(needs_layout_passes=False),
        name="sc_pair_gather",
    )(beta_f, bias_f, y_f)


def _combine_body(x_ref, g_ref, o_ref):
    xv = x_ref[...]
    bg = g_ref[0]
    bb = g_ref[1]
    t = jnp.log1p(xv) * bg + bb
    r = jax.nn.sigmoid(t) - jax.nn.sigmoid(bb)
    o_ref[...] = r.T


@functools.partial(jax.jit, static_argnames=("bm",))
def _tc_combine(x, g, *, bm):
    rows, ncols = x.shape
    return pl.pallas_call(
        _combine_body,
        grid=(rows // bm,),
        in_specs=[
            pl.BlockSpec((bm, ncols), lambda i: (i, 0)),
            pl.BlockSpec((2, bm, ncols), lambda i: (0, i, 0)),
        ],
        out_specs=pl.BlockSpec((ncols, bm), lambda i: (0, i)),
        out_shape=jax.ShapeDtypeStruct((ncols, rows), jnp.float32),
        name="tc_logsigm_combine",
    )(x, g)


def kernel(x, y, beta, bias):
    b, c = x.shape
    n_tab = beta.shape[1]
    yf = y.astype(jnp.int32).reshape(b * c)
    g = _sc_gather(beta.reshape(-1), bias.reshape(-1), yf,
                   n_tab=n_tab, ncols=c)
    return _tc_combine(x, g, bm=1024).T


# speedup vs baseline: 2.2270x; 1.2954x over previous
import functools

import jax
import jax.numpy as jnp
from jax import lax
from jax.experimental import pallas as pl
from jax.experimental.pallas import tpu as pltpu
from jax.experimental.pallas import tpu_sc as plsc

_LANES = 16
_NSUB = 16
_CROWS = 16
_GOFFS = tuple(range(0, 192, _LANES)) + (200 - _LANES,)


def _sc_gather_body(rows_per_sub, ncols, beta_hbm, bias_hbm, y_hbm, out_hbm,
                    table_v, idx0, idx1, idx2, idx3, val0, val1,
                    si0, si1, si2, si3, so0, so1):
    cid = lax.axis_index("c")
    sid = lax.axis_index("s")

    @pl.when(cid == 0)
    def _():
        pltpu.sync_copy(beta_hbm, table_v)

    @pl.when(cid != 0)
    def _():
        pltpu.sync_copy(bias_hbm, table_v)

    base = sid * rows_per_sub
    nch = rows_per_sub // _CROWS
    idx_b = (idx0, idx1, idx2, idx3)
    val_b = (val0, val1)
    si_b = (si0, si1, si2, si3)
    so_b = (so0, so1)

    def start_in(ch, b):
        pltpu.async_copy(y_hbm.at[pl.ds(base + ch * _CROWS, _CROWS), :],
                         idx_b[b], si_b[b])

    def wait_in(ch, b):
        pltpu.make_async_copy(y_hbm.at[pl.ds(base + ch * _CROWS, _CROWS), :],
                              idx_b[b], si_b[b]).wait()

    def start_out(ch, b):
        pltpu.async_copy(val_b[b],
                         out_hbm.at[cid, pl.ds(base + ch * _CROWS, _CROWS), :],
                         so_b[b])

    def wait_out(ch, b):
        pltpu.make_async_copy(
            val_b[b],
            out_hbm.at[cid, pl.ds(base + ch * _CROWS, _CROWS), :],
            so_b[b]).wait()

    def gather(ib, vb):
        iv_ref = idx_b[ib]
        ov_ref = val_b[vb]

        @plsc.parallel_loop(0, _CROWS, step=1, unroll=2)
        def _(r):
            ivs = [iv_ref[r, pl.ds(c, _LANES)] for c in _GOFFS]
            gs = [plsc.load_gather(table_v, [iv]) for iv in ivs]
            for k, c in enumerate(_GOFFS):
                ov_ref[r, pl.ds(c, _LANES)] = gs[k]

    start_in(0, 0)
    start_in(1, 1)
    start_in(2, 2)

    def body(h, _):
        for j in range(4):
            ch = 4 * h + j
            wait_in(ch, j)

            @pl.when(ch + 3 < nch)
            def _():
                start_in(ch + 3, (j + 3) % 4)

            if j >= 2:
                wait_out(ch - 2, j % 2)
            else:
                @pl.when(h > 0)
                def _():
                    wait_out(ch - 2, j % 2)

            gather(j, j % 2)
            start_out(ch, j % 2)
        return 0

    lax.fori_loop(0, nch // 4, body, 0)
    wait_out(nch - 2, 0)
    wait_out(nch - 1, 1)


@functools.partial(jax.jit, static_argnames=("n_tab",))
def _sc_gather(beta_f, bias_f, y2, *, n_tab):
    rows, ncols = y2.shape
    rows_per_sub = rows // _NSUB
    body = functools.partial(_sc_gather_body, rows_per_sub, ncols)
    return pl.kernel(
        body,
        out_type=jax.ShapeDtypeStruct((2, rows, ncols), jnp.float32),
        mesh=plsc.VectorSubcoreMesh(core_axis_name="c", subcore_axis_name="s"),
        scratch_types=[
            pltpu.VMEM((n_tab,), jnp.float32),
            pltpu.VMEM((_CROWS, ncols), jnp.int32),
            pltpu.VMEM((_CROWS, ncols), jnp.int32),
            pltpu.VMEM((_CROWS, ncols), jnp.int32),
            pltpu.VMEM((_CROWS, ncols), jnp.int32),
            pltpu.VMEM((_CROWS, ncols), jnp.float32),
            pltpu.VMEM((_CROWS, ncols), jnp.float32),
            pltpu.SemaphoreType.DMA,
            pltpu.SemaphoreType.DMA,
            pltpu.SemaphoreType.DMA,
            pltpu.SemaphoreType.DMA,
            pltpu.SemaphoreType.DMA,
            pltpu.SemaphoreType.DMA,
        ],
        compiler_params=pltpu.CompilerParams(needs_layout_passes=False),
        name="sc_pair_gather",
    )(beta_f, bias_f, y2)


def _combine_body(x_ref, g_ref, o_ref):
    xv = x_ref[...]
    bg = g_ref[0]
    bb = g_ref[1]
    t = jnp.log1p(xv) * bg + bb
    r = jax.nn.sigmoid(t) - jax.nn.sigmoid(bb)
    o_ref[...] = r.T


@functools.partial(jax.jit, static_argnames=("bm",))
def _tc_combine(x, g, *, bm):
    rows, ncols = x.shape
    return pl.pallas_call(
        _combine_body,
        grid=(rows // bm,),
        in_specs=[
            pl.BlockSpec((bm, ncols), lambda i: (i, 0)),
            pl.BlockSpec((2, bm, ncols), lambda i: (0, i, 0)),
        ],
        out_specs=pl.BlockSpec((ncols, bm), lambda i: (0, i)),
        out_shape=jax.ShapeDtypeStruct((ncols, rows), jnp.float32),
        name="tc_logsigm_combine",
    )(x, g)


def kernel(x, y, beta, bias):
    b, c = x.shape
    n_tab = beta.shape[1]
    del b, c
    y2 = y.astype(jnp.int32)
    g = _sc_gather(beta.reshape(-1), bias.reshape(-1), y2, n_tab=n_tab)
    return _tc_combine(x, g, bm=1024).T
